# Initial kernel scaffold; baseline (speedup 1.0000x reference)
import jax
import jax.numpy as jnp
from jax.experimental import pallas as pl


def kernel(x, emb, W, b):
    raise NotImplementedError("write your pallas kernel here")


# speedup vs baseline: 12.0758x; 12.0758x over previous
import functools

import jax
import jax.numpy as jnp
from jax import lax
from jax.experimental import pallas as pl
from jax.experimental.pallas import tpu as pltpu
from jax.experimental.pallas import tpu_sc as plsc

_VOCAB = 100000
_D = 128
_B = 4096
_SEQ = 200
_NCLS = 1000
_NCLS_PAD = 1024

_NC = 2
_NS = 16
_NW = _NC * _NS
_BPW = _B // _NW
_CHUNKS = ((0, 104), (104, 96))
_LANES = 16
_DCH = _D // _LANES


@functools.partial(
    pl.kernel,
    out_type=jax.ShapeDtypeStruct((_B, _D), jnp.float32),
    mesh=plsc.VectorSubcoreMesh(core_axis_name="c", subcore_axis_name="s"),
    scratch_types=[
        pltpu.VMEM((_BPW * _SEQ,), jnp.int32),
        pltpu.VMEM((2, _SEQ, _D), jnp.float32),
        pltpu.VMEM((_BPW, _D), jnp.float32),
        pltpu.SemaphoreType.DMA,
        pltpu.SemaphoreType.DMA,
    ],
)
def _sc_embed_sum(x_hbm, emb_hbm, out_hbm, idx_v, gbuf, obuf, sem0, sem1):
    wid = lax.axis_index("s") * _NC + lax.axis_index("c")
    base = wid * _BPW
    pltpu.sync_copy(x_hbm.at[pl.ds(base * _SEQ, _BPW * _SEQ)], idx_v)
    sems = (sem0, sem1)

    def gathers(i, b):
        return [
            pltpu.make_async_copy(
                emb_hbm.at[idx_v.at[pl.ds(i * _SEQ + off, ln)]],
                gbuf.at[b, pl.ds(off, ln)],
                sems[b],
            )
            for off, ln in _CHUNKS
        ]

    for c in gathers(0, 0):
        c.start()
    for c in gathers(1, 1):
        c.start()

    def row_body(i, b):
        for c in gathers(i, b):
            c.wait()

        def red(t, accs):
            return tuple(
                a + gbuf[b, t, pl.ds(k * _LANES, _LANES)]
                for k, a in enumerate(accs)
            )

        accs = lax.fori_loop(
            0, _SEQ, red,
            tuple(jnp.zeros((_LANES,), jnp.float32) for _ in range(_DCH)),
        )
        for k, a in enumerate(accs):
            obuf[i, pl.ds(k * _LANES, _LANES)] = a

        @pl.when(i + 2 < _BPW)
        def _():
            for c in gathers(i + 2, b):
                c.start()

    def pair_body(j, carry):
        row_body(2 * j, 0)
        row_body(2 * j + 1, 1)
        return carry

    lax.fori_loop(0, _BPW // 2, pair_body, 0)
    pltpu.sync_copy(obuf, out_hbm.at[pl.ds(base, _BPW)])


_BM = 512


def _mm_body(e_ref, w_ref, b_ref, o_ref):
    o_ref[...] = (
        lax.dot_general(
            e_ref[...],
            w_ref[...],
            dimension_numbers=(((1,), (1,)), ((), ())),
            preferred_element_type=jnp.float32,
        )
        * (1.0 / _SEQ)
        + b_ref[...]
    )


def _head_matmul(e_sum, w_pad, b_pad):
    return pl.pallas_call(
        _mm_body,
        grid=(_B // _BM,),
        in_specs=[
            pl.BlockSpec((_BM, _D), lambda i: (i, 0)),
            pl.BlockSpec((_NCLS_PAD, _D), lambda i: (0, 0)),
            pl.BlockSpec((1, _NCLS_PAD), lambda i: (0, 0)),
        ],
        out_specs=pl.BlockSpec((_BM, _NCLS_PAD), lambda i: (i, 0)),
        out_shape=jax.ShapeDtypeStruct((_B, _NCLS_PAD), jnp.float32),
    )(e_sum, w_pad, b_pad)


def kernel(x, emb, W, b):
    e_sum = _sc_embed_sum(x.reshape(-1), emb)
    w_pad = jnp.pad(W, ((0, _NCLS_PAD - _NCLS), (0, 0)))
    b_pad = jnp.pad(b, (0, _NCLS_PAD - _NCLS)).reshape(1, _NCLS_PAD)
    out = _head_matmul(e_sum, w_pad, b_pad)
    return out[:, :_NCLS]


# speedup vs baseline: 14.2652x; 1.1813x over previous
import functools

import jax
import jax.numpy as jnp
from jax import lax
from jax.experimental import pallas as pl
from jax.experimental.pallas import tpu as pltpu
from jax.experimental.pallas import tpu_sc as plsc

_VOCAB = 100000
_D = 128
_B = 4096
_SEQ = 200
_NCLS = 1000
_NCLS_PAD = 1024

_NC = 2
_NS = 16
_NW = _NC * _NS
_BPW = _B // _NW
_CHUNKS = ((0, 104), (104, 96))
_LANES = 16
_DCH = _D // _LANES


@functools.partial(
    pl.kernel,
    out_type=jax.ShapeDtypeStruct((_B, _D), jnp.float32),
    mesh=plsc.VectorSubcoreMesh(core_axis_name="c", subcore_axis_name="s"),
    scratch_types=[
        pltpu.VMEM((_BPW * _SEQ,), jnp.int32),
        pltpu.VMEM((3, _SEQ, _D), jnp.float32),
        pltpu.VMEM((_BPW, _D), jnp.float32),
        pltpu.SemaphoreType.DMA,
        pltpu.SemaphoreType.DMA,
        pltpu.SemaphoreType.DMA,
    ],
)
def _sc_embed_sum(x_hbm, emb_hbm, out_hbm, idx_v, gbuf, obuf, sem0, sem1, sem2):
    wid = lax.axis_index("s") * _NC + lax.axis_index("c")
    base = wid * _BPW
    pltpu.sync_copy(x_hbm.at[pl.ds(base * _SEQ, _BPW * _SEQ)], idx_v)
    sems = (sem0, sem1, sem2)
    nbuf = 3

    def gathers(i, b):
        return [
            pltpu.make_async_copy(
                emb_hbm.at[idx_v.at[pl.ds(i * _SEQ + off, ln)]],
                gbuf.at[b, pl.ds(off, ln)],
                sems[b],
            )
            for off, ln in _CHUNKS
        ]

    for b in range(nbuf):
        for c in gathers(b, b):
            c.start()

    def row_body(i, b):
        for c in gathers(i, b):
            c.wait()

        unroll = 4

        def red(j, accs):
            t = unroll * j
            return tuple(
                a
                + (
                    (
                        gbuf[b, t, pl.ds(k * _LANES, _LANES)]
                        + gbuf[b, t + 1, pl.ds(k * _LANES, _LANES)]
                    )
                    + (
                        gbuf[b, t + 2, pl.ds(k * _LANES, _LANES)]
                        + gbuf[b, t + 3, pl.ds(k * _LANES, _LANES)]
                    )
                )
                for k, a in enumerate(accs)
            )

        accs = lax.fori_loop(
            0, _SEQ // unroll, red,
            tuple(jnp.zeros((_LANES,), jnp.float32) for _ in range(_DCH)),
        )
        for k, a in enumerate(accs):
            obuf[i, pl.ds(k * _LANES, _LANES)] = a

        @pl.when(i + nbuf < _BPW)
        def _():
            for c in gathers(i + nbuf, b):
                c.start()

    def trip_body(j, carry):
        for b in range(nbuf):
            row_body(nbuf * j + b, b)
        return carry

    lax.fori_loop(0, _BPW // nbuf, trip_body, 0)
    row_body(_BPW - 2, (_BPW - 2) % nbuf)
    row_body(_BPW - 1, (_BPW - 1) % nbuf)
    pltpu.sync_copy(obuf, out_hbm.at[pl.ds(base, _BPW)])


_BM = 512


def _mm_body(e_ref, w_ref, b_ref, o_ref):
    o_ref[...] = (
        lax.dot_general(
            e_ref[...],
            w_ref[...],
            dimension_numbers=(((1,), (1,)), ((), ())),
            preferred_element_type=jnp.float32,
        )
        * (1.0 / _SEQ)
        + b_ref[...]
    )


def _head_matmul(e_sum, w, b2d):
    return pl.pallas_call(
        _mm_body,
        grid=(_B // _BM,),
        in_specs=[
            pl.BlockSpec((_BM, _D), lambda i: (i, 0)),
            pl.BlockSpec((_NCLS, _D), lambda i: (0, 0)),
            pl.BlockSpec((1, _NCLS), lambda i: (0, 0)),
        ],
        out_specs=pl.BlockSpec((_BM, _NCLS), lambda i: (i, 0)),
        out_shape=jax.ShapeDtypeStruct((_B, _NCLS), jnp.float32),
    )(e_sum, w, b2d)


def kernel(x, emb, W, b):
    e_sum = _sc_embed_sum(x.reshape(-1), emb)
    return _head_matmul(e_sum, W, b.reshape(1, _NCLS))


# speedup vs baseline: 15.5094x; 1.0872x over previous
import functools

import jax
import jax.numpy as jnp
from jax import lax
from jax.experimental import pallas as pl
from jax.experimental.pallas import tpu as pltpu
from jax.experimental.pallas import tpu_sc as plsc

_VOCAB = 100000
_D = 128
_B = 4096
_SEQ = 200
_NCLS = 1000
_NCLS_PAD = 1024

_NC = 2
_NS = 16
_NW = _NC * _NS
_BPW = _B // _NW
_CHUNKS = ((0, 104), (104, 96))
_LANES = 16
_DCH = _D // _LANES


@functools.partial(
    pl.kernel,
    out_type=jax.ShapeDtypeStruct((_B, _D), jnp.float32),
    mesh=plsc.VectorSubcoreMesh(core_axis_name="c", subcore_axis_name="s"),
    scratch_types=[
        pltpu.VMEM((_BPW * _SEQ,), jnp.int32),
        pltpu.VMEM((3, _SEQ, _D), jnp.float32),
        pltpu.VMEM((_BPW, _D), jnp.float32),
        pltpu.SemaphoreType.DMA,
        pltpu.SemaphoreType.DMA,
        pltpu.SemaphoreType.DMA,
    ],
)
def _sc_embed_sum(x_hbm, emb_hbm, out_hbm, idx_v, gbuf, obuf, sem0, sem1, sem2):
    wid = lax.axis_index("s") * _NC + lax.axis_index("c")
    base = wid * _BPW
    pltpu.sync_copy(x_hbm.at[pl.ds(base * _SEQ, _BPW * _SEQ)], idx_v)
    sems = (sem0, sem1, sem2)
    nbuf = 3

    def gathers(i, b):
        return [
            pltpu.make_async_copy(
                emb_hbm.at[idx_v.at[pl.ds(i * _SEQ + off, ln)]],
                gbuf.at[b, pl.ds(off, ln)],
                sems[b],
            )
            for off, ln in _CHUNKS
        ]

    for b in range(nbuf):
        for c in gathers(b, b):
            c.start()

    def row_body(i, b):
        for c in gathers(i, b):
            c.wait()

        unroll = 4

        def red(j, accs):
            t = unroll * j
            return tuple(
                a
                + (
                    (
                        gbuf[b, t, pl.ds(k * _LANES, _LANES)]
                        + gbuf[b, t + 1, pl.ds(k * _LANES, _LANES)]
                    )
                    + (
                        gbuf[b, t + 2, pl.ds(k * _LANES, _LANES)]
                        + gbuf[b, t + 3, pl.ds(k * _LANES, _LANES)]
                    )
                )
                for k, a in enumerate(accs)
            )

        accs = lax.fori_loop(
            0, _SEQ // unroll, red,
            tuple(jnp.zeros((_LANES,), jnp.float32) for _ in range(_DCH)),
        )
        for k, a in enumerate(accs):
            obuf[i, pl.ds(k * _LANES, _LANES)] = a

        @pl.when(i + nbuf < _BPW)
        def _():
            for c in gathers(i + nbuf, b):
                c.start()

    def trip_body(j, carry):
        for b in range(nbuf):
            row_body(nbuf * j + b, b)
        return carry

    lax.fori_loop(0, _BPW // nbuf, trip_body, 0)
    row_body(_BPW - 2, (_BPW - 2) % nbuf)
    row_body(_BPW - 1, (_BPW - 1) % nbuf)
    pltpu.sync_copy(obuf, out_hbm.at[pl.ds(base, _BPW)])


_BM = 512


def _mm_body(e_ref, w_ref, b_ref, o_ref):
    o_ref[...] = (
        lax.dot_general(
            w_ref[...],
            e_ref[...] * (1.0 / _SEQ),
            dimension_numbers=(((1,), (1,)), ((), ())),
            preferred_element_type=jnp.float32,
        )
        + b_ref[...]
    )


def _head_matmul(e_sum, w, bcol):
    return pl.pallas_call(
        _mm_body,
        grid=(_B // _BM,),
        in_specs=[
            pl.BlockSpec((_BM, _D), lambda i: (i, 0)),
            pl.BlockSpec((_NCLS, _D), lambda i: (0, 0)),
            pl.BlockSpec((_NCLS, 1), lambda i: (0, 0)),
        ],
        out_specs=pl.BlockSpec((_NCLS, _BM), lambda i: (0, i)),
        out_shape=jax.ShapeDtypeStruct((_NCLS, _B), jnp.float32),
    )(e_sum, w, bcol)


def kernel(x, emb, W, b):
    e_sum = _sc_embed_sum(x.reshape(-1), emb)
    out_t = _head_matmul(e_sum, W, b.reshape(_NCLS, 1))
    return jnp.transpose(out_t)


# speedup vs baseline: 15.9061x; 1.0256x over previous
import functools

import jax
import jax.numpy as jnp
from jax import lax
from jax.experimental import pallas as pl
from jax.experimental.pallas import tpu as pltpu
from jax.experimental.pallas import tpu_sc as plsc

_VOCAB = 100000
_D = 128
_B = 4096
_SEQ = 200
_NCLS = 1000
_NCLS_PAD = 1024

_NC = 2
_NS = 16
_NW = _NC * _NS
_BPW = _B // _NW
_CHUNKS = ((0, 104), (104, 96))
_LANES = 16
_DCH = _D // _LANES


@functools.partial(
    pl.kernel,
    out_type=jax.ShapeDtypeStruct((_B, _D), jnp.float32),
    mesh=plsc.VectorSubcoreMesh(core_axis_name="c", subcore_axis_name="s"),
    scratch_types=[
        pltpu.VMEM((_SEQ,), jnp.int32),
        pltpu.VMEM((_SEQ,), jnp.int32),
        pltpu.VMEM((_SEQ,), jnp.int32),
        pltpu.VMEM((3, _SEQ, _D), jnp.float32),
        pltpu.VMEM((_BPW, _D), jnp.float32),
        pltpu.SemaphoreType.DMA,
        pltpu.SemaphoreType.DMA,
        pltpu.SemaphoreType.DMA,
        pltpu.SemaphoreType.DMA,
        pltpu.SemaphoreType.DMA,
        pltpu.SemaphoreType.DMA,
    ],
)
def _sc_embed_sum(x_hbm, emb_hbm, out_hbm, ir0, ir1, ir2, gbuf, obuf,
                  sem0, sem1, sem2, isem0, isem1, isem2):
    wid = lax.axis_index("s") * _NC + lax.axis_index("c")
    base = wid * _BPW
    sems = (sem0, sem1, sem2)
    isems = (isem0, isem1, isem2)
    idxrows = (ir0, ir1, ir2)
    nbuf = 3

    def idxcopy(i, b):
        return pltpu.make_async_copy(x_hbm.at[base + i], idxrows[b], isems[b])

    def gathers(i, b):
        del i
        return [
            pltpu.make_async_copy(
                emb_hbm.at[idxrows[b].at[pl.ds(off, ln)]],
                gbuf.at[b, pl.ds(off, ln)],
                sems[b],
            )
            for off, ln in _CHUNKS
        ]

    for b in range(nbuf):
        idxcopy(b, b).start()
    for b in range(nbuf):
        idxcopy(b, b).wait()
        for c in gathers(b, b):
            c.start()

    def row_body(i, b):
        for c in gathers(i, b):
            c.wait()

        @pl.when(i + nbuf < _BPW)
        def _():
            idxcopy(i + nbuf, b).start()

        unroll = 4

        def red(j, accs):
            t = unroll * j
            return tuple(
                a
                + (
                    (
                        gbuf[b, t, pl.ds(k * _LANES, _LANES)]
                        + gbuf[b, t + 1, pl.ds(k * _LANES, _LANES)]
                    )
                    + (
                        gbuf[b, t + 2, pl.ds(k * _LANES, _LANES)]
                        + gbuf[b, t + 3, pl.ds(k * _LANES, _LANES)]
                    )
                )
                for k, a in enumerate(accs)
            )

        accs = lax.fori_loop(
            0, _SEQ // unroll, red,
            tuple(jnp.zeros((_LANES,), jnp.float32) for _ in range(_DCH)),
        )
        for k, a in enumerate(accs):
            obuf[i, pl.ds(k * _LANES, _LANES)] = a

        @pl.when(i + nbuf < _BPW)
        def _():
            idxcopy(i + nbuf, b).wait()
            for c in gathers(i + nbuf, b):
                c.start()

    def trip_body(j, carry):
        for b in range(nbuf):
            row_body(nbuf * j + b, b)
        return carry

    lax.fori_loop(0, _BPW // nbuf, trip_body, 0)
    row_body(_BPW - 2, (_BPW - 2) % nbuf)
    row_body(_BPW - 1, (_BPW - 1) % nbuf)
    pltpu.sync_copy(obuf, out_hbm.at[pl.ds(base, _BPW)])


_BM = 512


def _mm_body(e_ref, w_ref, b_ref, o_ref):
    o_ref[...] = (
        lax.dot_general(
            w_ref[...],
            e_ref[...] * (1.0 / _SEQ),
            dimension_numbers=(((1,), (1,)), ((), ())),
            preferred_element_type=jnp.float32,
        )
        + b_ref[...]
    )


def _head_matmul(e_sum, w, bcol):
    return pl.pallas_call(
        _mm_body,
        grid=(_B // _BM,),
        in_specs=[
            pl.BlockSpec((_BM, _D), lambda i: (i, 0)),
            pl.BlockSpec((_NCLS, _D), lambda i: (0, 0)),
            pl.BlockSpec((_NCLS, 1), lambda i: (0, 0)),
        ],
        out_specs=pl.BlockSpec((_NCLS, _BM), lambda i: (0, i)),
        out_shape=jax.ShapeDtypeStruct((_NCLS, _B), jnp.float32),
    )(e_sum, w, bcol)


def kernel(x, emb, W, b):
    e_sum = _sc_embed_sum(x, emb)
    out_t = _head_matmul(e_sum, W, b.reshape(_NCLS, 1))
    return jnp.transpose(out_t)
